# baseline (speedup 1.0000x reference)
import functools

import jax
import jax.numpy as jnp
from jax import lax
from jax.experimental import pallas as pl
from jax.experimental.pallas import tpu as pltpu
from jax.experimental.pallas import tpu_sc as plsc

D = 128
F = 78
AUX = 24
NG = 10000
NM = 5882
NB = 9

NMP = 6144
NGP = 12288
NEP = 36864
NG2 = 10240
E2P = 3 * NG2
E1P = 12288
EH = NEP // 2
TRASH = NMP - 1

_F32 = jnp.float32


def _mm(a, w):
    return jnp.dot(a, w, preferred_element_type=_F32)


def _layernorm(x, g, b):
    mu = jnp.mean(x, axis=-1, keepdims=True)
    xc = x - mu
    var = jnp.mean(xc * xc, axis=-1, keepdims=True)
    return xc / jnp.sqrt(var + 1e-5) * g + b


def _row(i):
    return (i, 0)


def _const0(i):
    return (0, 0)


def _off(k):
    def f(i):
        return (i + k, 0)

    return f


def _stage(rows, R, terms, b1, w2, b2, ln_gb, residual_term, projs):
    grid_n = rows // R
    assert grid_n * R == rows and R % 8 == 0

    arrays = []
    specs = []
    term_layout = []
    for entries, w in terms:
        idxs = []
        modes = []
        for arr, imap, mode in entries:
            idxs.append(len(arrays))
            modes.append(mode)
            arrays.append(arr)
            br = R // 6 if mode == "rep6" else (3 * R if mode == "sum3" else R)
            specs.append(pl.BlockSpec((br, arr.shape[1]), imap))
        term_layout.append((idxs, modes, w is not None))
    weights = [w for _, w in terms if w is not None]
    consts = weights + [b1, w2, b2]
    if ln_gb is not None:
        consts += [ln_gb[0], ln_gb[1]]
    consts += list(projs)
    for c in consts:
        arrays.append(c)
        specs.append(pl.BlockSpec(c.shape, _const0))

    n_in = len(arrays)
    n_out = 1 + len(projs)
    has_ln = ln_gb is not None
    n_w = len(weights)
    const_base = sum(len(t[0]) for t in term_layout)

    def _entry(ref, mode):
        a = ref[...]
        if mode == "rep6":
            a = jnp.broadcast_to(a[:, None, :], (R // 6, 6, D)).reshape(R, D)
        elif mode == "sum3":
            a = a.reshape(R, 3, D).sum(axis=1)
        return a

    def body(*refs):
        in_refs = refs[:n_in]
        out_refs = refs[n_in:]
        wi = 0
        acc = None
        term_first_block = []
        for idxs, modes, has_w in term_layout:
            a = _entry(in_refs[idxs[0]], modes[0])
            term_first_block.append(a)
            for j, m in zip(idxs[1:], modes[1:]):
                a = a + _entry(in_refs[j], m)
            if has_w:
                a = _mm(a, in_refs[const_base + wi][...])
                wi += 1
            acc = a if acc is None else acc + a
        k = const_base + n_w
        x = acc + in_refs[k][...]
        h = jax.nn.silu(x)
        y = _mm(h, in_refs[k + 1][...]) + in_refs[k + 2][...]
        k += 3
        if has_ln:
            y = _layernorm(y, in_refs[k][...], in_refs[k + 1][...])
            k += 2
        if residual_term is not None:
            y = term_first_block[residual_term] + y
        out_refs[0][...] = y
        for pi in range(n_out - 1):
            out_refs[pi + 1][...] = _mm(y, in_refs[k + pi][...])

    res = pl.pallas_call(
        body,
        grid=(grid_n,),
        in_specs=specs,
        out_specs=[pl.BlockSpec((R, D), _row) for _ in range(n_out)],
        out_shape=[jax.ShapeDtypeStruct((rows, D), _F32) for _ in range(n_out)],
    )(*arrays)
    return res


def _decode_stage(h_out, feat, wa, wb, b1, w2, b2, R):
    grid_n = NG // R

    def body(h_ref, f_ref, wa_r, wb_r, b1_r, w2_r, b2_r, y_ref, sy_ref, sf_ref):
        i = pl.program_id(0)
        f = f_ref[...]
        x = _mm(h_ref[...], wa_r[...]) + _mm(f, wb_r[...]) + b1_r[...]
        y = _mm(jax.nn.silu(x), w2_r[...]) + b2_r[...]
        y_ref[...] = y
        sy = jnp.sum(y, axis=0, keepdims=True)
        sf = jnp.sum(f, axis=0, keepdims=True)

        @pl.when(i == 0)
        def _():
            sy_ref[...] = sy
            sf_ref[...] = sf

        @pl.when(i > 0)
        def _():
            sy_ref[...] += sy
            sf_ref[...] += sf

    return pl.pallas_call(
        body,
        grid=(grid_n,),
        in_specs=[
            pl.BlockSpec((R, D), _row),
            pl.BlockSpec((R, D), _row),
            pl.BlockSpec((D, D), _const0),
            pl.BlockSpec((D, D), _const0),
            pl.BlockSpec((1, D), _const0),
            pl.BlockSpec((D, D), _const0),
            pl.BlockSpec((1, D), _const0),
        ],
        out_specs=[
            pl.BlockSpec((R, D), _row),
            pl.BlockSpec((1, D), _const0),
            pl.BlockSpec((1, D), _const0),
        ],
        out_shape=[
            jax.ShapeDtypeStruct((NG, D), _F32),
            jax.ShapeDtypeStruct((1, D), _F32),
            jax.ShapeDtypeStruct((1, D), _F32),
        ],
    )(h_out, feat, wa, wb, b1, w2, b2)


def _apply_conservation(y, sy, sf, R):
    grid_n = NG // R

    def body(y_ref, sy_ref, sf_ref, o_ref):
        d = (sf_ref[...] - sy_ref[...]) * (1.0 / NG)
        o_ref[...] = (y_ref[...] + d)[:, :F]

    return pl.pallas_call(
        body,
        grid=(grid_n,),
        in_specs=[
            pl.BlockSpec((R, D), _row),
            pl.BlockSpec((1, D), _const0),
            pl.BlockSpec((1, D), _const0),
        ],
        out_specs=pl.BlockSpec((R, F), _row),
        out_shape=jax.ShapeDtypeStruct((NG, F), _F32),
    )(y, sy, sf)


_SC_MESH = dict(core_axis_name="c", subcore_axis_name="s")


def _gather_rows(table, idx_rs, e_pad, n_chunks, chunk):
    per_w = n_chunks * chunk
    nbuf = min(3, n_chunks)
    stripe = NMP // 16
    mesh = plsc.VectorSubcoreMesh(**_SC_MESH)

    @functools.partial(
        pl.kernel,
        mesh=mesh,
        out_type=jax.ShapeDtypeStruct((e_pad, D), _F32),
        scratch_types=[
            pltpu.VMEM_SHARED((NMP, D), _F32),
            pltpu.VMEM((n_chunks, chunk), jnp.int32),
        ]
        + [pltpu.VMEM((chunk, D), _F32)] * nbuf
        + [pltpu.SemaphoreType.DMA] * (2 * nbuf),
    )
    def k(table_hbm, idx_hbm, out_hbm, table_s, idx_v, *bufsem):
        bufs = bufsem[:nbuf]
        sg = bufsem[nbuf : 2 * nbuf]
        ss = bufsem[2 * nbuf :]
        cid = lax.axis_index("c")
        sid = lax.axis_index("s")
        wid = sid * 2 + cid
        base = wid * per_w
        pltpu.sync_copy(
            table_hbm.at[pl.ds(sid * stripe, stripe)],
            table_s.at[pl.ds(sid * stripe, stripe)],
        )
        pltpu.sync_copy(idx_hbm.at[wid], idx_v)
        plsc.subcore_barrier()
        g_d = [None] * n_chunks
        s_d = [None] * n_chunks
        for j in range(nbuf):
            g_d[j] = pltpu.async_copy(table_s.at[idx_v.at[j]], bufs[j], sg[j])
        for j in range(n_chunks):
            s = j % nbuf
            g_d[j].wait()
            s_d[j] = pltpu.async_copy(
                bufs[s], out_hbm.at[pl.ds(base + j * chunk, chunk)], ss[s]
            )
            if j + nbuf < n_chunks:
                s_d[j].wait()
                g_d[j + nbuf] = pltpu.async_copy(
                    table_s.at[idx_v.at[j + nbuf]], bufs[s], sg[s]
                )
        for j in range(max(0, n_chunks - nbuf), n_chunks):
            s_d[j].wait()

    return k(table, idx_rs)


def _scatter_add_rows(data, idx_rs, e_pad, n_chunks, chunk):
    per_w = n_chunks * chunk
    stripe = NMP // 16
    mesh = plsc.VectorSubcoreMesh(**_SC_MESH)
    zeros = jnp.zeros((stripe, D), _F32)

    nbuf = min(3, n_chunks)

    @functools.partial(
        pl.kernel,
        mesh=mesh,
        out_type=jax.ShapeDtypeStruct((2, NMP, D), _F32),
        scratch_types=[
            pltpu.VMEM_SHARED((NMP, D), _F32),
            pltpu.VMEM((n_chunks, chunk), jnp.int32),
        ]
        + [pltpu.VMEM((chunk, D), _F32)] * nbuf
        + [pltpu.SemaphoreType.DMA] * (2 * nbuf),
    )
    def k(data_hbm, idx_hbm, zeros_hbm, out_hbm, table_s, idx_v, *bufsem):
        bufs = bufsem[:nbuf]
        sl = bufsem[nbuf : 2 * nbuf]
        sa = bufsem[2 * nbuf :]
        cid = lax.axis_index("c")
        sid = lax.axis_index("s")
        wid = sid * 2 + cid
        base = wid * per_w
        l_d = [None] * n_chunks
        a_d = [None] * n_chunks
        for j in range(nbuf):
            l_d[j] = pltpu.async_copy(
                data_hbm.at[pl.ds(base + j * chunk, chunk)], bufs[j], sl[j]
            )
        pltpu.sync_copy(idx_hbm.at[wid], idx_v)
        pltpu.sync_copy(zeros_hbm, table_s.at[pl.ds(sid * stripe, stripe)])
        plsc.subcore_barrier()
        for j in range(n_chunks):
            s = j % nbuf
            l_d[j].wait()
            a_d[j] = pltpu.async_copy(
                bufs[s], table_s.at[idx_v.at[j]], sa[s], add=True
            )
            if j + nbuf < n_chunks:
                a_d[j].wait()
                l_d[j + nbuf] = pltpu.async_copy(
                    data_hbm.at[pl.ds(base + (j + nbuf) * chunk, chunk)],
                    bufs[s],
                    sl[s],
                )
        for j in range(max(0, n_chunks - nbuf), n_chunks):
            a_d[j].wait()
        plsc.subcore_barrier()
        pltpu.sync_copy(
            table_s.at[pl.ds(sid * stripe, stripe)],
            out_hbm.at[cid, pl.ds(sid * stripe, stripe)],
        )

    return k(data, idx_rs, zeros)


def _pad_rows(a, rows):
    return jnp.pad(a, ((0, rows - a.shape[0]), (0, 0)))


def _pad_cols(a, w=D):
    return jnp.pad(a, ((0, 0), (0, w - a.shape[1])))


def _split_w(w, *sizes):
    out, o = [], 0
    for s in sizes:
        out.append(w[o : o + s])
        o += s
    return out


def _prep(p):
    w1, w2 = p["ws"]
    b1 = p["bs"][0].reshape(1, D)
    b2 = p["bs"][1].reshape(1, -1)
    ln = None
    if "ln_g" in p:
        ln = (p["ln_g"].reshape(1, D), p["ln_b"].reshape(1, D))
    return w1, b1, w2, b2, ln


def kernel(features, params, g2m_src, g2m_dst, mesh_src, mesh_dst, m2g_src, m2g_dst):
    del g2m_src, mesh_src, m2g_dst
    p = params
    feat = features[0]

    feat_full = _pad_rows(_pad_cols(feat), NGP)
    feat78 = _pad_cols(feat[:, :F])

    g2m_attr = _pad_cols(_pad_rows(p["g2m_attr"], NGP), 8)
    g2m_dst_p = jnp.concatenate(
        [g2m_dst, jnp.full((E1P - NG,), TRASH, jnp.int32)]
    ).reshape(32, 3, 128)

    mesh_attr = _pad_cols(_pad_rows(p["mesh_attr"], NEP), 8)
    mdst_pad = jnp.concatenate(
        [mesh_dst, jnp.full((NEP - NM * 6,), TRASH, jnp.int32)]
    )
    mdst_h = [mdst_pad[h * EH : (h + 1) * EH].reshape(32, 6, 96) for h in range(2)]

    m2g_attr = _pad_cols(_pad_rows(p["m2g_attr"], E2P), 8)
    msrc_rs = jnp.concatenate(
        [m2g_src, jnp.zeros((E2P - 3 * NG,), jnp.int32)]
    ).reshape(32, 8, 120)

    g2m_wa, g2m_wb = _split_w(p["g2m_msg"]["ws"][0], D, D)
    blk_w = []
    for blk in p["blocks"]:
        e1a, e1b, e1c = _split_w(blk["edge"]["ws"][0], D, D, D)
        n1a, n1b = _split_w(blk["node"]["ws"][0], D, D)
        blk_w.append((e1a, e1b, e1c, n1a, n1b))
    v1a, v1b = _split_w(p["m2g_msg"]["ws"][0], D, D)
    u1a, u1b = _split_w(p["m2g_upd"]["ws"][0], D, D)
    da, db_raw = _split_w(p["decode"]["ws"][0], D, F)
    db = jnp.pad(db_raw, ((0, D - F), (0, 0)))
    dec_w2 = jnp.pad(p["decode"]["ws"][1], ((0, 0), (0, D - F)))
    dec_b2 = jnp.pad(p["decode"]["bs"][1], (0, D - F)).reshape(1, D)
    dec_b1 = p["decode"]["bs"][0].reshape(1, D)

    w1m, b1m, w2m, b2m, lnm = _prep(p["mesh_edge_enc"])
    w1m = jnp.pad(w1m, ((0, 8 - 3), (0, 0)))
    em = [
        _stage(EH, 2304, [([(mesh_attr, _off(h * 8), None)], w1m)],
               b1m, w2m, b2m, lnm, None, [])[0]
        for h in range(2)
    ]

    w1, b1, w2, b2, ln = _prep(p["enc_node"])
    w1 = jnp.pad(w1, ((0, D - F - AUX), (0, 0)))
    (h_grid,) = _stage(NGP, 2048, [([(feat_full, _row, None)], w1)], b1, w2, b2, ln, None, [])

    w1, b1, w2, b2, ln = _prep(p["enc_edge_g2m"])
    w1 = jnp.pad(w1, ((0, 8 - 3), (0, 0)))
    (e_g2m,) = _stage(NGP, 2048, [([(g2m_attr, _row, None)], w1)], b1, w2, b2, ln, None, [])

    _, b1, w2, b2, ln = _prep(p["g2m_msg"])
    (msg,) = _stage(
        NGP, 2048,
        [([(e_g2m, _row, None)], g2m_wa), ([(h_grid, _row, None)], g2m_wb)],
        b1, w2, b2, ln, None, [],
    )

    pg = _scatter_add_rows(msg, g2m_dst_p, E1P, 3, 128)

    w1u, b1u, w2u, b2u, lnu = _prep(p["g2m_upd"])
    h_mesh, hs, hd = _stage(
        NMP, 2048,
        [([(pg[0], _row, None), (pg[1], _row, None)], w1u)],
        b1u, w2u, b2u, lnu, None,
        [blk_w[0][1], blk_w[0][2]],
    )

    for bi in range(NB):
        e1a = blk_w[bi][0]
        n1a, n1b = blk_w[bi][3], blk_w[bi][4]
        _, eb1, ew2, eb2, eln = _prep(p["blocks"][bi]["edge"])
        g = [_gather_rows(hd, mdst_h[h], EH, 6, 96) for h in range(2)]
        em = [
            _stage(
                EH, 2304,
                [([(em[h], _row, None)], e1a),
                 ([(hs, _off(h * 8), "rep6"), (g[h], _row, None)], None)],
                eb1, ew2, eb2, eln, 0, [],
            )[0]
            for h in range(2)
        ]
        pm = [_scatter_add_rows(em[h], mdst_h[h], EH, 6, 96) for h in range(2)]
        _, nb1, nw2, nb2, nln = _prep(p["blocks"][bi]["node"])
        if bi + 1 < NB:
            nxt = [blk_w[bi + 1][1], blk_w[bi + 1][2]]
        else:
            nxt = [v1b, v1b]
        h_mesh, hs, hd = _stage(
            NMP, 2048,
            [([(h_mesh, _row, None)], n1a),
             ([(pm[0][0], _row, None), (pm[0][1], _row, None),
               (pm[1][0], _row, None), (pm[1][1], _row, None)], n1b)],
            nb1, nw2, nb2, nln, 0, nxt,
        )

    w1d, b1d, w2d, b2d, lnd = _prep(p["dec_edge"])
    w1d = jnp.pad(w1d, ((0, 8 - 3), (0, 0)))
    (ed,) = _stage(E2P, 2048, [([(m2g_attr, _row, None)], w1d)], b1d, w2d, b2d, lnd, None, [])

    g2 = _gather_rows(hd, msrc_rs, E2P, 8, 120)

    _, b1v, w2v, b2v, lnv = _prep(p["m2g_msg"])
    (msg2,) = _stage(
        E2P, 2048,
        [([(ed, _row, None)], v1a), ([(g2, _row, None)], None)],
        b1v, w2v, b2v, lnv, None, [],
    )

    _, b1g, w2g, b2g, lng = _prep(p["m2g_upd"])
    (h_out,) = _stage(
        NG2, 1024,
        [
            ([(h_grid, _row, None)], u1a),
            ([(msg2, _row, "sum3")], u1b),
        ],
        b1g, w2g, b2g, lng, None, [],
    )

    y, sy, sf = _decode_stage(h_out, feat78, da, db, dec_b1, dec_w2, dec_b2, 2000)
    out = _apply_conservation(y, sy, sf, 2000)
    return out[None]


# speedup vs baseline: 1.1123x; 1.1123x over previous
import functools

import jax
import jax.numpy as jnp
from jax import lax
from jax.experimental import pallas as pl
from jax.experimental.pallas import tpu as pltpu
from jax.experimental.pallas import tpu_sc as plsc

D = 128
F = 78
AUX = 24
NG = 10000
NM = 5882
NB = 9

NMP = 6144
NGP = 12288
NEP = 36864
NG2 = 10240
E2P = 3 * NG2
E1P = 12288
EH = NEP // 2
TRASH = NMP - 1

_F32 = jnp.float32


def _mm(a, w):
    return jnp.dot(a, w, preferred_element_type=_F32)


def _layernorm(x, g, b):
    mu = jnp.mean(x, axis=-1, keepdims=True)
    xc = x - mu
    var = jnp.mean(xc * xc, axis=-1, keepdims=True)
    return xc / jnp.sqrt(var + 1e-5) * g + b


def _row(i):
    return (i, 0)


def _const0(i):
    return (0, 0)


def _off(k):
    def f(i):
        return (i + k, 0)

    return f


def _p3(c):
    def f(i):
        return (c, i, 0)

    return f


def _stage(rows, R, terms, b1, w2, b2, ln_gb, residual_term, projs):
    grid_n = rows // R
    assert grid_n * R == rows and R % 8 == 0

    arrays = []
    specs = []
    term_layout = []
    for entries, w in terms:
        idxs = []
        modes = []
        for arr, imap, mode in entries:
            idxs.append(len(arrays))
            arrays.append(arr)
            br = R // 6 if mode == "rep6" else (3 * R if mode == "sum3" else R)
            if arr.ndim == 3:
                modes.append((mode, True))
                specs.append(pl.BlockSpec((1, br, arr.shape[2]), imap))
            else:
                modes.append((mode, False))
                specs.append(pl.BlockSpec((br, arr.shape[1]), imap))
        term_layout.append((idxs, modes, w is not None))
    weights = [w for _, w in terms if w is not None]
    consts = weights + [b1, w2, b2]
    if ln_gb is not None:
        consts += [ln_gb[0], ln_gb[1]]
    consts += list(projs)
    for c in consts:
        arrays.append(c)
        specs.append(pl.BlockSpec(c.shape, _const0))

    n_in = len(arrays)
    n_out = 1 + len(projs)
    has_ln = ln_gb is not None
    n_w = len(weights)
    const_base = sum(len(t[0]) for t in term_layout)

    def _entry(ref, mode_3d):
        mode, is3d = mode_3d
        a = ref[0] if is3d else ref[...]
        if mode == "rep6":
            a = jnp.broadcast_to(a[:, None, :], (R // 6, 6, D)).reshape(R, D)
        elif mode == "sum3":
            a = a.reshape(R, 3, D).sum(axis=1)
        return a

    def body(*refs):
        in_refs = refs[:n_in]
        out_refs = refs[n_in:]
        wi = 0
        acc = None
        term_first_block = []
        for idxs, modes, has_w in term_layout:
            a = _entry(in_refs[idxs[0]], modes[0])
            term_first_block.append(a)
            for j, m in zip(idxs[1:], modes[1:]):
                a = a + _entry(in_refs[j], m)
            if has_w:
                a = _mm(a, in_refs[const_base + wi][...])
                wi += 1
            acc = a if acc is None else acc + a
        k = const_base + n_w
        x = acc + in_refs[k][...]
        h = jax.nn.silu(x)
        y = _mm(h, in_refs[k + 1][...]) + in_refs[k + 2][...]
        k += 3
        if has_ln:
            y = _layernorm(y, in_refs[k][...], in_refs[k + 1][...])
            k += 2
        if residual_term is not None:
            y = term_first_block[residual_term] + y
        out_refs[0][...] = y
        for pi in range(n_out - 1):
            out_refs[pi + 1][...] = _mm(y, in_refs[k + pi][...])

    res = pl.pallas_call(
        body,
        grid=(grid_n,),
        in_specs=specs,
        out_specs=[pl.BlockSpec((R, D), _row) for _ in range(n_out)],
        out_shape=[jax.ShapeDtypeStruct((rows, D), _F32) for _ in range(n_out)],
    )(*arrays)
    return res


def _decode_stage(h_out, feat, wa, wb, b1, w2, b2, R):
    grid_n = NG // R

    def body(h_ref, f_ref, wa_r, wb_r, b1_r, w2_r, b2_r, y_ref, sy_ref, sf_ref):
        i = pl.program_id(0)
        f = f_ref[...]
        x = _mm(h_ref[...], wa_r[...]) + _mm(f, wb_r[...]) + b1_r[...]
        y = _mm(jax.nn.silu(x), w2_r[...]) + b2_r[...]
        y_ref[...] = y
        sy = jnp.sum(y, axis=0, keepdims=True)
        sf = jnp.sum(f, axis=0, keepdims=True)

        @pl.when(i == 0)
        def _():
            sy_ref[...] = sy
            sf_ref[...] = sf

        @pl.when(i > 0)
        def _():
            sy_ref[...] += sy
            sf_ref[...] += sf

    return pl.pallas_call(
        body,
        grid=(grid_n,),
        in_specs=[
            pl.BlockSpec((R, D), _row),
            pl.BlockSpec((R, D), _row),
            pl.BlockSpec((D, D), _const0),
            pl.BlockSpec((D, D), _const0),
            pl.BlockSpec((1, D), _const0),
            pl.BlockSpec((D, D), _const0),
            pl.BlockSpec((1, D), _const0),
        ],
        out_specs=[
            pl.BlockSpec((R, D), _row),
            pl.BlockSpec((1, D), _const0),
            pl.BlockSpec((1, D), _const0),
        ],
        out_shape=[
            jax.ShapeDtypeStruct((NG, D), _F32),
            jax.ShapeDtypeStruct((1, D), _F32),
            jax.ShapeDtypeStruct((1, D), _F32),
        ],
    )(h_out, feat, wa, wb, b1, w2, b2)


def _apply_conservation(y, sy, sf, R):
    grid_n = NG // R

    def body(y_ref, sy_ref, sf_ref, o_ref):
        d = (sf_ref[...] - sy_ref[...]) * (1.0 / NG)
        o_ref[...] = (y_ref[...] + d)[:, :F]

    return pl.pallas_call(
        body,
        grid=(grid_n,),
        in_specs=[
            pl.BlockSpec((R, D), _row),
            pl.BlockSpec((1, D), _const0),
            pl.BlockSpec((1, D), _const0),
        ],
        out_specs=pl.BlockSpec((R, F), _row),
        out_shape=jax.ShapeDtypeStruct((NG, F), _F32),
    )(y, sy, sf)


_SC_MESH = dict(core_axis_name="c", subcore_axis_name="s")


def _gather_rows(table, idx_rs, e_pad, n_chunks, chunk):
    per_w = n_chunks * chunk
    nbuf = min(3, n_chunks)
    stripe = NMP // 16
    mesh = plsc.VectorSubcoreMesh(**_SC_MESH)

    @functools.partial(
        pl.kernel,
        mesh=mesh,
        out_type=jax.ShapeDtypeStruct((e_pad, D), _F32),
        scratch_types=[
            pltpu.VMEM_SHARED((NMP, D), _F32),
            pltpu.VMEM((n_chunks, chunk), jnp.int32),
        ]
        + [pltpu.VMEM((chunk, D), _F32)] * nbuf
        + [pltpu.SemaphoreType.DMA] * (2 * nbuf),
    )
    def k(table_hbm, idx_hbm, out_hbm, table_s, idx_v, *bufsem):
        bufs = bufsem[:nbuf]
        sg = bufsem[nbuf : 2 * nbuf]
        ss = bufsem[2 * nbuf :]
        cid = lax.axis_index("c")
        sid = lax.axis_index("s")
        wid = sid * 2 + cid
        base = wid * per_w
        pltpu.sync_copy(
            table_hbm.at[pl.ds(sid * stripe, stripe)],
            table_s.at[pl.ds(sid * stripe, stripe)],
        )
        pltpu.sync_copy(idx_hbm.at[wid], idx_v)
        plsc.subcore_barrier()
        g_d = [None] * n_chunks
        s_d = [None] * n_chunks
        for j in range(nbuf):
            g_d[j] = pltpu.async_copy(table_s.at[idx_v.at[j]], bufs[j], sg[j])
        for j in range(n_chunks):
            s = j % nbuf
            g_d[j].wait()
            s_d[j] = pltpu.async_copy(
                bufs[s], out_hbm.at[pl.ds(base + j * chunk, chunk)], ss[s]
            )
            if j + nbuf < n_chunks:
                s_d[j].wait()
                g_d[j + nbuf] = pltpu.async_copy(
                    table_s.at[idx_v.at[j + nbuf]], bufs[s], sg[s]
                )
        for j in range(max(0, n_chunks - nbuf), n_chunks):
            s_d[j].wait()

    return k(table, idx_rs)


def _scatter_add_rows(data, idx_rs, e_pad, n_chunks, chunk):
    per_w = n_chunks * chunk
    stripe = NMP // 16
    mesh = plsc.VectorSubcoreMesh(**_SC_MESH)
    zeros = jnp.zeros((stripe, D), _F32)

    nbuf = min(3, n_chunks)

    @functools.partial(
        pl.kernel,
        mesh=mesh,
        out_type=jax.ShapeDtypeStruct((2, NMP, D), _F32),
        scratch_types=[
            pltpu.VMEM_SHARED((NMP, D), _F32),
            pltpu.VMEM((n_chunks, chunk), jnp.int32),
        ]
        + [pltpu.VMEM((chunk, D), _F32)] * nbuf
        + [pltpu.SemaphoreType.DMA] * (2 * nbuf),
    )
    def k(data_hbm, idx_hbm, zeros_hbm, out_hbm, table_s, idx_v, *bufsem):
        bufs = bufsem[:nbuf]
        sl = bufsem[nbuf : 2 * nbuf]
        sa = bufsem[2 * nbuf :]
        cid = lax.axis_index("c")
        sid = lax.axis_index("s")
        wid = sid * 2 + cid
        base = wid * per_w
        l_d = [None] * n_chunks
        a_d = [None] * n_chunks
        for j in range(nbuf):
            l_d[j] = pltpu.async_copy(
                data_hbm.at[pl.ds(base + j * chunk, chunk)], bufs[j], sl[j]
            )
        pltpu.sync_copy(idx_hbm.at[wid], idx_v)
        pltpu.sync_copy(zeros_hbm, table_s.at[pl.ds(sid * stripe, stripe)])
        plsc.subcore_barrier()
        for j in range(n_chunks):
            s = j % nbuf
            l_d[j].wait()
            a_d[j] = pltpu.async_copy(
                bufs[s], table_s.at[idx_v.at[j]], sa[s], add=True
            )
            if j + nbuf < n_chunks:
                a_d[j].wait()
                l_d[j + nbuf] = pltpu.async_copy(
                    data_hbm.at[pl.ds(base + (j + nbuf) * chunk, chunk)],
                    bufs[s],
                    sl[s],
                )
        for j in range(max(0, n_chunks - nbuf), n_chunks):
            a_d[j].wait()
        plsc.subcore_barrier()
        pltpu.sync_copy(
            table_s.at[pl.ds(sid * stripe, stripe)],
            out_hbm.at[cid, pl.ds(sid * stripe, stripe)],
        )

    return k(data, idx_rs, zeros)


def _pad_rows(a, rows):
    return jnp.pad(a, ((0, rows - a.shape[0]), (0, 0)))


def _pad_cols(a, w=D):
    return jnp.pad(a, ((0, 0), (0, w - a.shape[1])))


def _split_w(w, *sizes):
    out, o = [], 0
    for s in sizes:
        out.append(w[o : o + s])
        o += s
    return out


def _prep(p):
    w1, w2 = p["ws"]
    b1 = p["bs"][0].reshape(1, D)
    b2 = p["bs"][1].reshape(1, -1)
    ln = None
    if "ln_g" in p:
        ln = (p["ln_g"].reshape(1, D), p["ln_b"].reshape(1, D))
    return w1, b1, w2, b2, ln


def kernel(features, params, g2m_src, g2m_dst, mesh_src, mesh_dst, m2g_src, m2g_dst):
    del g2m_src, mesh_src, m2g_dst
    p = params
    feat = features[0]

    feat_full = _pad_rows(_pad_cols(feat), NGP)
    feat78 = _pad_cols(feat[:, :F])

    g2m_attr = _pad_cols(_pad_rows(p["g2m_attr"], NGP), 8)
    g2m_dst_p = jnp.concatenate(
        [g2m_dst, jnp.full((E1P - NG,), TRASH, jnp.int32)]
    ).reshape(32, 3, 128)

    mesh_attr = _pad_cols(_pad_rows(p["mesh_attr"], NEP), 8)
    mdst_rs = jnp.concatenate(
        [mesh_dst, jnp.full((NEP - NM * 6,), TRASH, jnp.int32)]
    ).reshape(32, 9, 128)

    m2g_attr = _pad_cols(_pad_rows(p["m2g_attr"], E2P), 8)
    msrc_rs = jnp.concatenate(
        [m2g_src, jnp.zeros((E2P - 3 * NG,), jnp.int32)]
    ).reshape(32, 8, 120)

    g2m_wa, g2m_wb = _split_w(p["g2m_msg"]["ws"][0], D, D)
    blk_w = []
    for blk in p["blocks"]:
        e1a, e1b, e1c = _split_w(blk["edge"]["ws"][0], D, D, D)
        n1a, n1b = _split_w(blk["node"]["ws"][0], D, D)
        blk_w.append((e1a, e1b, e1c, n1a, n1b))
    v1a, v1b = _split_w(p["m2g_msg"]["ws"][0], D, D)
    u1a, u1b = _split_w(p["m2g_upd"]["ws"][0], D, D)
    da, db_raw = _split_w(p["decode"]["ws"][0], D, F)
    db = jnp.pad(db_raw, ((0, D - F), (0, 0)))
    dec_w2 = jnp.pad(p["decode"]["ws"][1], ((0, 0), (0, D - F)))
    dec_b2 = jnp.pad(p["decode"]["bs"][1], (0, D - F)).reshape(1, D)
    dec_b1 = p["decode"]["bs"][0].reshape(1, D)

    w1m, b1m, w2m, b2m, lnm = _prep(p["mesh_edge_enc"])
    w1m = jnp.pad(w1m, ((0, 8 - 3), (0, 0)))
    (em,) = _stage(NEP, 2304, [([(mesh_attr, _row, None)], w1m)], b1m, w2m, b2m, lnm, None, [])

    w1, b1, w2, b2, ln = _prep(p["enc_node"])
    w1 = jnp.pad(w1, ((0, D - F - AUX), (0, 0)))
    (h_grid,) = _stage(NGP, 2048, [([(feat_full, _row, None)], w1)], b1, w2, b2, ln, None, [])

    w1, b1, w2, b2, ln = _prep(p["enc_edge_g2m"])
    w1 = jnp.pad(w1, ((0, 8 - 3), (0, 0)))
    (e_g2m,) = _stage(NGP, 2048, [([(g2m_attr, _row, None)], w1)], b1, w2, b2, ln, None, [])

    _, b1, w2, b2, ln = _prep(p["g2m_msg"])
    (msg,) = _stage(
        NGP, 2048,
        [([(e_g2m, _row, None)], g2m_wa), ([(h_grid, _row, None)], g2m_wb)],
        b1, w2, b2, ln, None, [],
    )

    pg = _scatter_add_rows(msg, g2m_dst_p, E1P, 3, 128)

    w1u, b1u, w2u, b2u, lnu = _prep(p["g2m_upd"])
    h_mesh, hs, hd = _stage(
        NMP, 2048,
        [([(pg, _p3(0), None), (pg, _p3(1), None)], w1u)],
        b1u, w2u, b2u, lnu, None,
        [blk_w[0][1], blk_w[0][2]],
    )

    for bi in range(NB):
        e1a = blk_w[bi][0]
        n1a, n1b = blk_w[bi][3], blk_w[bi][4]
        _, eb1, ew2, eb2, eln = _prep(p["blocks"][bi]["edge"])
        g = _gather_rows(hd, mdst_rs, NEP, 9, 128)
        (em,) = _stage(
            NEP, 2304,
            [([(em, _row, None)], e1a), ([(hs, _row, "rep6"), (g, _row, None)], None)],
            eb1, ew2, eb2, eln, 0, [],
        )
        pm = _scatter_add_rows(em, mdst_rs, NEP, 9, 128)
        _, nb1, nw2, nb2, nln = _prep(p["blocks"][bi]["node"])
        if bi + 1 < NB:
            nxt = [blk_w[bi + 1][1], blk_w[bi + 1][2]]
        else:
            nxt = [v1b, v1b]
        h_mesh, hs, hd = _stage(
            NMP, 2048,
            [([(h_mesh, _row, None)], n1a),
             ([(pm, _p3(0), None), (pm, _p3(1), None)], n1b)],
            nb1, nw2, nb2, nln, 0, nxt,
        )

    w1d, b1d, w2d, b2d, lnd = _prep(p["dec_edge"])
    w1d = jnp.pad(w1d, ((0, 8 - 3), (0, 0)))
    (ed,) = _stage(E2P, 2048, [([(m2g_attr, _row, None)], w1d)], b1d, w2d, b2d, lnd, None, [])

    g2 = _gather_rows(hd, msrc_rs, E2P, 8, 120)

    _, b1v, w2v, b2v, lnv = _prep(p["m2g_msg"])
    (msg2,) = _stage(
        E2P, 2048,
        [([(ed, _row, None)], v1a), ([(g2, _row, None)], None)],
        b1v, w2v, b2v, lnv, None, [],
    )

    _, b1g, w2g, b2g, lng = _prep(p["m2g_upd"])
    (h_out,) = _stage(
        NG2, 1024,
        [
            ([(h_grid, _row, None)], u1a),
            ([(msg2, _row, "sum3")], u1b),
        ],
        b1g, w2g, b2g, lng, None, [],
    )

    y, sy, sf = _decode_stage(h_out, feat78, da, db, dec_b1, dec_w2, dec_b2, 2000)
    out = _apply_conservation(y, sy, sf, 2000)
    return out[None]


# speedup vs baseline: 1.1143x; 1.0018x over previous
import functools

import jax
import jax.numpy as jnp
from jax import lax
from jax.experimental import pallas as pl
from jax.experimental.pallas import tpu as pltpu
from jax.experimental.pallas import tpu_sc as plsc

D = 128
F = 78
AUX = 24
NG = 10000
NM = 5882
NB = 9

NMP = 6144
NGP = 12288
NEP = 36864
NG2 = 10240
E2P = 3 * NG2
E1P = 12288
EH = NEP // 2
TRASH = NMP - 1

_F32 = jnp.float32


def _mm(a, w):
    return jnp.dot(a, w, preferred_element_type=_F32)


def _layernorm(x, g, b):
    mu = jnp.mean(x, axis=-1, keepdims=True)
    xc = x - mu
    var = jnp.mean(xc * xc, axis=-1, keepdims=True)
    return xc / jnp.sqrt(var + 1e-5) * g + b


def _row(i):
    return (i, 0)


def _const0(i):
    return (0, 0)


def _off(k):
    def f(i):
        return (i + k, 0)

    return f


def _p3(c):
    def f(i):
        return (c, i, 0)

    return f


def _stage(rows, R, terms, b1, w2, b2, ln_gb, residual_term, projs):
    grid_n = rows // R
    assert grid_n * R == rows and R % 8 == 0

    arrays = []
    specs = []
    term_layout = []
    for entries, w in terms:
        idxs = []
        modes = []
        for arr, imap, mode in entries:
            idxs.append(len(arrays))
            arrays.append(arr)
            br = R // 6 if mode == "rep6" else (3 * R if mode == "sum3" else R)
            if arr.ndim == 3:
                modes.append((mode, True))
                specs.append(pl.BlockSpec((1, br, arr.shape[2]), imap))
            else:
                modes.append((mode, False))
                specs.append(pl.BlockSpec((br, arr.shape[1]), imap))
        term_layout.append((idxs, modes, w is not None))
    weights = [w for _, w in terms if w is not None]
    consts = weights + [b1, w2, b2]
    if ln_gb is not None:
        consts += [ln_gb[0], ln_gb[1]]
    consts += list(projs)
    for c in consts:
        arrays.append(c)
        specs.append(pl.BlockSpec(c.shape, _const0))

    n_in = len(arrays)
    n_out = 1 + len(projs)
    has_ln = ln_gb is not None
    n_w = len(weights)
    const_base = sum(len(t[0]) for t in term_layout)

    def _entry(ref, mode_3d):
        mode, is3d = mode_3d
        a = ref[0] if is3d else ref[...]
        if mode == "rep6":
            a = jnp.broadcast_to(a[:, None, :], (R // 6, 6, D)).reshape(R, D)
        elif mode == "sum3":
            a = a.reshape(R, 3, D).sum(axis=1)
        return a

    def body(*refs):
        in_refs = refs[:n_in]
        out_refs = refs[n_in:]
        wi = 0
        acc = None
        term_first_block = []
        for idxs, modes, has_w in term_layout:
            a = _entry(in_refs[idxs[0]], modes[0])
            term_first_block.append(a)
            for j, m in zip(idxs[1:], modes[1:]):
                a = a + _entry(in_refs[j], m)
            if has_w:
                a = _mm(a, in_refs[const_base + wi][...])
                wi += 1
            acc = a if acc is None else acc + a
        k = const_base + n_w
        x = acc + in_refs[k][...]
        h = jax.nn.silu(x)
        y = _mm(h, in_refs[k + 1][...]) + in_refs[k + 2][...]
        k += 3
        if has_ln:
            y = _layernorm(y, in_refs[k][...], in_refs[k + 1][...])
            k += 2
        if residual_term is not None:
            y = term_first_block[residual_term] + y
        out_refs[0][...] = y
        for pi in range(n_out - 1):
            out_refs[pi + 1][...] = _mm(y, in_refs[k + pi][...])

    res = pl.pallas_call(
        body,
        grid=(grid_n,),
        in_specs=specs,
        out_specs=[pl.BlockSpec((R, D), _row) for _ in range(n_out)],
        out_shape=[jax.ShapeDtypeStruct((rows, D), _F32) for _ in range(n_out)],
    )(*arrays)
    return res


def _decode_stage(h_out, feat, wa, wb, b1, w2, b2, R):
    grid_n = NG // R

    def body(h_ref, f_ref, wa_r, wb_r, b1_r, w2_r, b2_r, y_ref, sy_ref, sf_ref):
        i = pl.program_id(0)
        f = f_ref[...]
        x = _mm(h_ref[...], wa_r[...]) + _mm(f, wb_r[...]) + b1_r[...]
        y = _mm(jax.nn.silu(x), w2_r[...]) + b2_r[...]
        y_ref[...] = y
        sy = jnp.sum(y, axis=0, keepdims=True)
        sf = jnp.sum(f, axis=0, keepdims=True)

        @pl.when(i == 0)
        def _():
            sy_ref[...] = sy
            sf_ref[...] = sf

        @pl.when(i > 0)
        def _():
            sy_ref[...] += sy
            sf_ref[...] += sf

    return pl.pallas_call(
        body,
        grid=(grid_n,),
        in_specs=[
            pl.BlockSpec((R, D), _row),
            pl.BlockSpec((R, D), _row),
            pl.BlockSpec((D, D), _const0),
            pl.BlockSpec((D, D), _const0),
            pl.BlockSpec((1, D), _const0),
            pl.BlockSpec((D, D), _const0),
            pl.BlockSpec((1, D), _const0),
        ],
        out_specs=[
            pl.BlockSpec((R, D), _row),
            pl.BlockSpec((1, D), _const0),
            pl.BlockSpec((1, D), _const0),
        ],
        out_shape=[
            jax.ShapeDtypeStruct((NG, D), _F32),
            jax.ShapeDtypeStruct((1, D), _F32),
            jax.ShapeDtypeStruct((1, D), _F32),
        ],
    )(h_out, feat, wa, wb, b1, w2, b2)


def _apply_conservation(y, sy, sf, R):
    grid_n = NG // R

    def body(y_ref, sy_ref, sf_ref, o_ref):
        d = (sf_ref[...] - sy_ref[...]) * (1.0 / NG)
        o_ref[...] = (y_ref[...] + d)[:, :F]

    return pl.pallas_call(
        body,
        grid=(grid_n,),
        in_specs=[
            pl.BlockSpec((R, D), _row),
            pl.BlockSpec((1, D), _const0),
            pl.BlockSpec((1, D), _const0),
        ],
        out_specs=pl.BlockSpec((R, F), _row),
        out_shape=jax.ShapeDtypeStruct((NG, F), _F32),
    )(y, sy, sf)


_SC_MESH = dict(core_axis_name="c", subcore_axis_name="s")


def _gather_rows(table, idx_rs, e_pad, n_chunks, chunk):
    per_w = n_chunks * chunk
    nbuf = min(3, n_chunks)
    stripe = NMP // 16
    mesh = plsc.VectorSubcoreMesh(**_SC_MESH)

    @functools.partial(
        pl.kernel,
        mesh=mesh,
        out_type=jax.ShapeDtypeStruct((e_pad, D), _F32),
        scratch_types=[
            pltpu.VMEM_SHARED((NMP, D), _F32),
            pltpu.VMEM((n_chunks, chunk), jnp.int32),
        ]
        + [pltpu.VMEM((chunk, D), _F32)] * nbuf
        + [pltpu.SemaphoreType.DMA] * (2 * nbuf),
    )
    def k(table_hbm, idx_hbm, out_hbm, table_s, idx_v, *bufsem):
        bufs = bufsem[:nbuf]
        sg = bufsem[nbuf : 2 * nbuf]
        ss = bufsem[2 * nbuf :]
        cid = lax.axis_index("c")
        sid = lax.axis_index("s")
        wid = sid * 2 + cid
        base = wid * per_w
        pltpu.sync_copy(
            table_hbm.at[pl.ds(sid * stripe, stripe)],
            table_s.at[pl.ds(sid * stripe, stripe)],
        )
        pltpu.sync_copy(idx_hbm.at[wid], idx_v)
        plsc.subcore_barrier()
        g_d = [None] * n_chunks
        s_d = [None] * n_chunks
        for j in range(nbuf):
            g_d[j] = pltpu.async_copy(table_s.at[idx_v.at[j]], bufs[j], sg[j])
        for j in range(n_chunks):
            s = j % nbuf
            g_d[j].wait()
            s_d[j] = pltpu.async_copy(
                bufs[s], out_hbm.at[pl.ds(base + j * chunk, chunk)], ss[s]
            )
            if j + nbuf < n_chunks:
                s_d[j].wait()
                g_d[j + nbuf] = pltpu.async_copy(
                    table_s.at[idx_v.at[j + nbuf]], bufs[s], sg[s]
                )
        for j in range(max(0, n_chunks - nbuf), n_chunks):
            s_d[j].wait()

    return k(table, idx_rs)


def _scatter_add_rows(data, idx_rs, e_pad, n_chunks, chunk):
    per_w = n_chunks * chunk
    stripe = NMP // 16
    mesh = plsc.VectorSubcoreMesh(**_SC_MESH)
    zeros = jnp.zeros((stripe, D), _F32)

    nbuf = min(3, n_chunks)

    @functools.partial(
        pl.kernel,
        mesh=mesh,
        out_type=jax.ShapeDtypeStruct((2, NMP, D), _F32),
        scratch_types=[
            pltpu.VMEM_SHARED((NMP, D), _F32),
            pltpu.VMEM((n_chunks, chunk), jnp.int32),
        ]
        + [pltpu.VMEM((chunk, D), _F32)] * nbuf
        + [pltpu.SemaphoreType.DMA] * (2 * nbuf),
    )
    def k(data_hbm, idx_hbm, zeros_hbm, out_hbm, table_s, idx_v, *bufsem):
        bufs = bufsem[:nbuf]
        sl = bufsem[nbuf : 2 * nbuf]
        sa = bufsem[2 * nbuf :]
        cid = lax.axis_index("c")
        sid = lax.axis_index("s")
        wid = sid * 2 + cid
        base = wid * per_w
        l_d = [None] * n_chunks
        a_d = [None] * n_chunks
        for j in range(nbuf):
            l_d[j] = pltpu.async_copy(
                data_hbm.at[pl.ds(base + j * chunk, chunk)], bufs[j], sl[j]
            )
        pltpu.sync_copy(idx_hbm.at[wid], idx_v)
        pltpu.sync_copy(zeros_hbm, table_s.at[pl.ds(sid * stripe, stripe)])
        plsc.subcore_barrier()
        for j in range(n_chunks):
            s = j % nbuf
            l_d[j].wait()
            a_d[j] = pltpu.async_copy(
                bufs[s], table_s.at[idx_v.at[j]], sa[s], add=True
            )
            if j + nbuf < n_chunks:
                a_d[j].wait()
                l_d[j + nbuf] = pltpu.async_copy(
                    data_hbm.at[pl.ds(base + (j + nbuf) * chunk, chunk)],
                    bufs[s],
                    sl[s],
                )
        for j in range(max(0, n_chunks - nbuf), n_chunks):
            a_d[j].wait()
        plsc.subcore_barrier()
        pltpu.sync_copy(
            table_s.at[pl.ds(sid * stripe, stripe)],
            out_hbm.at[cid, pl.ds(sid * stripe, stripe)],
        )

    return k(data, idx_rs, zeros)


def _pad_rows(a, rows):
    return jnp.pad(a, ((0, rows - a.shape[0]), (0, 0)))


def _pad_cols(a, w=D):
    return jnp.pad(a, ((0, 0), (0, w - a.shape[1])))


def _split_w(w, *sizes):
    out, o = [], 0
    for s in sizes:
        out.append(w[o : o + s])
        o += s
    return out


def _prep(p):
    w1, w2 = p["ws"]
    b1 = p["bs"][0].reshape(1, D)
    b2 = p["bs"][1].reshape(1, -1)
    ln = None
    if "ln_g" in p:
        ln = (p["ln_g"].reshape(1, D), p["ln_b"].reshape(1, D))
    return w1, b1, w2, b2, ln


def kernel(features, params, g2m_src, g2m_dst, mesh_src, mesh_dst, m2g_src, m2g_dst):
    del g2m_src, mesh_src, m2g_dst
    p = params
    feat = features[0]

    feat_full = _pad_rows(_pad_cols(feat), NGP)
    feat78 = _pad_cols(feat[:, :F])

    g2m_attr = _pad_rows(p["g2m_attr"], NGP)
    g2m_dst_p = jnp.concatenate(
        [g2m_dst, jnp.full((E1P - NG,), TRASH, jnp.int32)]
    ).reshape(32, 3, 128)

    mesh_attr = _pad_rows(p["mesh_attr"], NEP)
    mdst_rs = jnp.concatenate(
        [mesh_dst, jnp.full((NEP - NM * 6,), TRASH, jnp.int32)]
    ).reshape(32, 9, 128)

    m2g_attr = _pad_rows(p["m2g_attr"], E2P)
    msrc_rs = jnp.concatenate(
        [m2g_src, jnp.zeros((E2P - 3 * NG,), jnp.int32)]
    ).reshape(32, 8, 120)

    g2m_wa, g2m_wb = _split_w(p["g2m_msg"]["ws"][0], D, D)
    blk_w = []
    for blk in p["blocks"]:
        e1a, e1b, e1c = _split_w(blk["edge"]["ws"][0], D, D, D)
        n1a, n1b = _split_w(blk["node"]["ws"][0], D, D)
        blk_w.append((e1a, e1b, e1c, n1a, n1b))
    v1a, v1b = _split_w(p["m2g_msg"]["ws"][0], D, D)
    u1a, u1b = _split_w(p["m2g_upd"]["ws"][0], D, D)
    da, db_raw = _split_w(p["decode"]["ws"][0], D, F)
    db = jnp.pad(db_raw, ((0, D - F), (0, 0)))
    dec_w2 = jnp.pad(p["decode"]["ws"][1], ((0, 0), (0, D - F)))
    dec_b2 = jnp.pad(p["decode"]["bs"][1], (0, D - F)).reshape(1, D)
    dec_b1 = p["decode"]["bs"][0].reshape(1, D)

    w1m, b1m, w2m, b2m, lnm = _prep(p["mesh_edge_enc"])
    (em,) = _stage(NEP, 2304, [([(mesh_attr, _row, None)], w1m)], b1m, w2m, b2m, lnm, None, [])

    w1, b1, w2, b2, ln = _prep(p["enc_node"])
    w1 = jnp.pad(w1, ((0, D - F - AUX), (0, 0)))
    (h_grid,) = _stage(NGP, 2048, [([(feat_full, _row, None)], w1)], b1, w2, b2, ln, None, [])

    w1, b1, w2, b2, ln = _prep(p["enc_edge_g2m"])
    (e_g2m,) = _stage(NGP, 2048, [([(g2m_attr, _row, None)], w1)], b1, w2, b2, ln, None, [])

    _, b1, w2, b2, ln = _prep(p["g2m_msg"])
    (msg,) = _stage(
        NGP, 2048,
        [([(e_g2m, _row, None)], g2m_wa), ([(h_grid, _row, None)], g2m_wb)],
        b1, w2, b2, ln, None, [],
    )

    pg = _scatter_add_rows(msg, g2m_dst_p, E1P, 3, 128)

    w1u, b1u, w2u, b2u, lnu = _prep(p["g2m_upd"])
    h_mesh, hs, hd = _stage(
        NMP, 2048,
        [([(pg, _p3(0), None), (pg, _p3(1), None)], w1u)],
        b1u, w2u, b2u, lnu, None,
        [blk_w[0][1], blk_w[0][2]],
    )

    for bi in range(NB):
        e1a = blk_w[bi][0]
        n1a, n1b = blk_w[bi][3], blk_w[bi][4]
        _, eb1, ew2, eb2, eln = _prep(p["blocks"][bi]["edge"])
        g = _gather_rows(hd, mdst_rs, NEP, 9, 128)
        (em,) = _stage(
            NEP, 2304,
            [([(em, _row, None)], e1a), ([(hs, _row, "rep6"), (g, _row, None)], None)],
            eb1, ew2, eb2, eln, 0, [],
        )
        pm = _scatter_add_rows(em, mdst_rs, NEP, 9, 128)
        _, nb1, nw2, nb2, nln = _prep(p["blocks"][bi]["node"])
        if bi + 1 < NB:
            nxt = [blk_w[bi + 1][1], blk_w[bi + 1][2]]
        else:
            nxt = [v1b, v1b]
        h_mesh, hs, hd = _stage(
            NMP, 2048,
            [([(h_mesh, _row, None)], n1a),
             ([(pm, _p3(0), None), (pm, _p3(1), None)], n1b)],
            nb1, nw2, nb2, nln, 0, nxt,
        )

    w1d, b1d, w2d, b2d, lnd = _prep(p["dec_edge"])
    (ed,) = _stage(E2P, 2048, [([(m2g_attr, _row, None)], w1d)], b1d, w2d, b2d, lnd, None, [])

    g2 = _gather_rows(hd, msrc_rs, E2P, 8, 120)

    _, b1v, w2v, b2v, lnv = _prep(p["m2g_msg"])
    (msg2,) = _stage(
        E2P, 2048,
        [([(ed, _row, None)], v1a), ([(g2, _row, None)], None)],
        b1v, w2v, b2v, lnv, None, [],
    )

    _, b1g, w2g, b2g, lng = _prep(p["m2g_upd"])
    (h_out,) = _stage(
        NG2, 1024,
        [
            ([(h_grid, _row, None)], u1a),
            ([(msg2, _row, "sum3")], u1b),
        ],
        b1g, w2g, b2g, lng, None, [],
    )

    y, sy, sf = _decode_stage(h_out, feat78, da, db, dec_b1, dec_w2, dec_b2, 2000)
    out = _apply_conservation(y, sy, sf, 2000)
    return out[None]
